# Initial kernel scaffold; baseline (speedup 1.0000x reference)
import jax
import jax.numpy as jnp
from jax.experimental import pallas as pl


def kernel(input_y):
    raise NotImplementedError("write your pallas kernel here")


# speedup vs baseline: 3.1315x; 3.1315x over previous
import functools

import numpy as np
import jax
import jax.numpy as jnp
from jax import lax
from jax.experimental import pallas as pl
from jax.experimental.pallas import tpu as pltpu

SIGMA = 25.0
P = 7
N_PIX = P * P
K1 = 18
K2 = 55
WIN = 37
V = WIN // 2
NO = WIN * WIN
STEP = 4
H = 224
W = 224
Hc = H - P + 1
Wc = W - P + 1
L = Hc * Wc
NG = 56
B = NG * NG
DCONST = (SIGMA ** 2) * N_PIX
NPAD = 64

def _grid_1d():
    g = list(range(0, H - P + 1, STEP))
    if (H - P + 1) % STEP != 1:
        g.append(H - P)
    return np.array(g, np.int32)

_G1 = _grid_1d()
_HRS = np.tile(_G1, NG).astype(np.int32)
_WRS = np.repeat(_G1, NG).astype(np.int32)


def _grid_select_rows(d):
    c = d.shape[1]
    dp = jnp.concatenate([d, jnp.zeros((4 * NG - Hc, c), jnp.float32)], axis=0)
    r = dp.reshape(NG, 4, c)
    return jnp.concatenate([r[0:NG - 1, 0, :], r[NG - 2:NG - 1, 1, :]], axis=0)


def _bm_body(ip_ref, out_ref, sp_scr):
    o = pl.program_id(0)

    @pl.when(o == 0)
    def _():
        ipsq = ip_ref[...] * ip_ref[...]
        s = ipsq[0:254, :]
        for i in range(1, P):
            s = s + ipsq[i:254 + i, :]
        s2 = s[:, 0:254]
        for i in range(1, P):
            s2 = s2 + s[:, i:254 + i]
        sp_scr[...] = s2

    ir = o // WIN
    ic = o % WIN
    img = ip_ref[V:V + H, V:V + W]
    ri = lax.broadcasted_iota(jnp.int32, (H, H + 2 * V), 0)
    rj = lax.broadcasted_iota(jnp.int32, (H, H + 2 * V), 1)
    rsh = (rj == ri + ir).astype(jnp.float32)
    li = lax.broadcasted_iota(jnp.int32, (W + 2 * V, W), 0)
    lj = lax.broadcasted_iota(jnp.int32, (W + 2 * V, W), 1)
    csh = (li == lj + ic).astype(jnp.float32)
    tmp = jnp.dot(rsh, ip_ref[...], preferred_element_type=jnp.float32)
    ish = jnp.dot(tmp, csh, preferred_element_type=jnp.float32)
    prod = img * ish
    s = prod[0:Hc, :]
    for i in range(1, P):
        s = s + prod[i:Hc + i, :]
    co = s[:, 0:Wc]
    for i in range(1, P):
        co = co + s[:, i:Wc + i]
    sq = H + 2 * V - P + 1
    ri2 = lax.broadcasted_iota(jnp.int32, (Hc, sq), 0)
    rj2 = lax.broadcasted_iota(jnp.int32, (Hc, sq), 1)
    rsh2 = (rj2 == ri2 + ir).astype(jnp.float32)
    li2 = lax.broadcasted_iota(jnp.int32, (sq, Wc), 0)
    lj2 = lax.broadcasted_iota(jnp.int32, (sq, Wc), 1)
    csh2 = (li2 == lj2 + ic).astype(jnp.float32)
    tmp2 = jnp.dot(rsh2, sp_scr[...], preferred_element_type=jnp.float32)
    ns = jnp.dot(tmp2, csh2, preferred_element_type=jnp.float32)
    dfull = (ns - 2.0 * co) * (1.0 / N_PIX)
    rsel = _grid_select_rows(dfull)
    gt = _grid_select_rows(rsel.T)
    out_ref[0] = gt


def _block_match_dist(img):
    ip = jnp.pad(img, V)
    return pl.pallas_call(
        _bm_body,
        grid=(NO,),
        in_specs=[
            pl.BlockSpec((H + 2 * V, W + 2 * V), lambda o: (0, 0)),
        ],
        out_specs=pl.BlockSpec((1, NG, NG), lambda o: (o, 0, 0)),
        out_shape=jax.ShapeDtypeStruct((NO, NG, NG), jnp.float32),
        scratch_shapes=[pltpu.VMEM((H + 2 * V - P + 1, W + 2 * V - P + 1), jnp.float32)],
    )(ip).reshape(NO, B)


def _topk_body(k, dist_ref, hrs_ref, wrs_ref, out_ref, d_scr):
    oi = lax.broadcasted_iota(jnp.int32, (NO, B), 0)
    ir = oi // WIN - V
    ic = oi % WIN - V
    hr = hrs_ref[...]
    wr = wrs_ref[...]
    ch = hr + ir
    cw = wr + ic
    oob = (ch < 0) | (ch > H - P) | (cw < 0) | (cw > W - P)
    d = jnp.where(oob, jnp.inf, dist_ref[...])
    d = jnp.where(oi == V * WIN + V, -jnp.inf, d)
    d_scr[...] = d

    def step(j, _):
        dc = d_scr[...]
        m = jnp.min(dc, axis=0, keepdims=True)
        sel = jnp.min(jnp.where(dc == m, oi, NO), axis=0, keepdims=True)
        gi = (hr + (sel // WIN - V)) * Wc + (wr + (sel % WIN - V))
        out_ref[pl.ds(j, 1), :] = gi
        d_scr[...] = jnp.where(oi == sel, jnp.inf, dc)
        return 0

    lax.fori_loop(0, k, step, 0)


def _topk_indices(dist, k):
    hrs = jnp.asarray(_HRS).reshape(1, B)
    wrs = jnp.asarray(_WRS).reshape(1, B)
    return pl.pallas_call(
        functools.partial(_topk_body, k),
        in_specs=[
            pl.BlockSpec((NO, B), lambda: (0, 0)),
            pl.BlockSpec((1, B), lambda: (0, 0)),
            pl.BlockSpec((1, B), lambda: (0, 0)),
        ],
        out_specs=pl.BlockSpec((k, B), lambda: (0, 0)),
        out_shape=jax.ShapeDtypeStruct((k, B), jnp.int32),
        scratch_shapes=[pltpu.VMEM((NO, B), jnp.float32)],
    )(dist, hrs, wrs)


def _denoise_body(k, gb, two_q, yref, *rest):
    if two_q:
        xref, out_ref, a_scr = rest
    else:
        out_ref, a_scr = rest
    yblk = yref[...].reshape(gb, k, NPAD)
    qsrc = xref[...].reshape(gb, k, NPAD) if two_q else yblk
    eye3 = (lax.broadcasted_iota(jnp.int32, (gb, k, k), 1)
            == lax.broadcasted_iota(jnp.int32, (gb, k, k), 2)).astype(jnp.float32)
    for i in range(gb):
        qi = lax.dot_general(qsrc[i], qsrc[i], (((1,), (1,)), ((), ())),
                             preferred_element_type=jnp.float32)
        if two_q:
            qi = qi + DCONST * eye3[0]
        a_scr[pl.ds(i * k, k), :] = qi

    i1 = lax.broadcasted_iota(jnp.int32, (gb, k, k), 1)
    i2 = lax.broadcasted_iota(jnp.int32, (gb, k, k), 2)

    def gj_step(j, _):
        a = a_scr[...].reshape(gb, k, k)
        mrow = (i1 == j)
        mcol = (i2 == j)
        mjj = mrow & mcol
        p = jnp.sum(jnp.where(mjj, a, 0.0), axis=(1, 2), keepdims=True)
        rp = 1.0 / p
        colv = jnp.sum(jnp.where(mcol, a, 0.0), axis=2, keepdims=True)
        roww = jnp.sum(jnp.where(mrow, a, 0.0), axis=1, keepdims=True)
        anew = a - colv * roww * rp
        anew = jnp.where(mrow, roww * rp, anew)
        anew = jnp.where(mcol, -colv * rp, anew)
        anew = jnp.where(mjj, rp, anew)
        a_scr[...] = anew.reshape(gb * k, k)
        return 0

    lax.fori_loop(0, k, gj_step, 0)

    qinv = a_scr[...].reshape(gb, k, k)
    theta = eye3 - DCONST * qinv
    wts = 1.0 / jnp.clip(jnp.sum(theta * theta, axis=2, keepdims=True),
                         1.0 / k, 1.0)
    lane = lax.broadcasted_iota(jnp.int32, (gb, k, NPAD), 2)
    for i in range(gb):
        xh = jnp.dot(theta[i], yblk[i], preferred_element_type=jnp.float32)
        xw = xh * wts[i]
        xw = jnp.where(lane[0] == N_PIX, wts[i], xw)
        out_ref[pl.ds(i * k, k), :] = xw


def _denoise_rows(yrows, k, xrows=None):
    gb = 16 if k == K1 else 8
    steps = B // gb
    two_q = xrows is not None
    ins = [yrows] if not two_q else [yrows, xrows]
    in_specs = [pl.BlockSpec((gb * k, NPAD), lambda g: (g, 0))] * len(ins)
    return pl.pallas_call(
        functools.partial(_denoise_body, k, gb, two_q),
        grid=(steps,),
        in_specs=in_specs,
        out_specs=pl.BlockSpec((gb * k, NPAD), lambda g: (g, 0)),
        out_shape=jax.ShapeDtypeStruct((B * k, NPAD), jnp.float32),
        scratch_shapes=[pltpu.VMEM((gb * k, k), jnp.float32)],
    )(*ins)


def _fold_body(x_ref, out_ref, num_scr, den_scr):
    num_scr[...] = jnp.zeros((H, W), jnp.float32)
    den_scr[...] = jnp.zeros((H, W), jnp.float32)
    wplane = x_ref[N_PIX]
    for a in range(P):
        for b in range(P):
            num_scr[a:a + Hc, b:b + Wc] += x_ref[a * P + b]
            den_scr[a:a + Hc, b:b + Wc] += wplane
    out_ref[...] = num_scr[...] / den_scr[...]


def _fold_divide(xsh):
    return pl.pallas_call(
        _fold_body,
        in_specs=[pl.BlockSpec((NPAD, Hc, Wc), lambda: (0, 0, 0))],
        out_specs=pl.BlockSpec((H, W), lambda: (0, 0)),
        out_shape=jax.ShapeDtypeStruct((H, W), jnp.float32),
        scratch_shapes=[pltpu.VMEM((H, W), jnp.float32),
                        pltpu.VMEM((H, W), jnp.float32)],
    )(xsh)


def _unfold_pad(img):
    cols = [img[a:a + Hc, b:b + Wc] for a in range(P) for b in range(P)]
    u = jnp.stack(cols, axis=-1).reshape(L, N_PIX)
    return jnp.concatenate([u, jnp.zeros((L, NPAD - N_PIX), jnp.float32)], axis=1)


def _gather_rows(tables, gi, k):
    return tuple(t[gi] for t in tables)


def _scatter_acc(rows, gi):
    acc = jnp.zeros((L, NPAD), jnp.float32)
    return acc.at[gi].add(rows)


def _one_pass(img_for_match, k, tables):
    dist = _block_match_dist(img_for_match)
    idx = _topk_indices(dist, k)
    gi = idx.T.reshape(B * k)
    gathered = _gather_rows(tables, gi, k)
    if len(gathered) == 1:
        rows = _denoise_rows(gathered[0], k)
    else:
        rows = _denoise_rows(gathered[0], k, gathered[1])
    acc = _scatter_acc(rows, gi)
    xsh = acc.T.reshape(NPAD, Hc, Wc)
    return _fold_divide(xsh)


def kernel(input_y):
    img = input_y[0, 0]
    u_y = _unfold_pad(img)
    x1 = _one_pass(img, K1, (u_y,))
    u_x1 = _unfold_pad(x1)
    x2 = _one_pass(x1, K2, (u_y, u_x1))
    return x2.reshape(1, 1, H, W)
